# Initial kernel scaffold; baseline (speedup 1.0000x reference)
import jax
import jax.numpy as jnp
from jax.experimental import pallas as pl


def kernel(x_nchw, conv_w, conv_b, bn_gamma, bn_beta, prelu_a):
    raise NotImplementedError("write your pallas kernel here")


# speedup vs baseline: 1.1559x; 1.1559x over previous
import jax
import jax.numpy as jnp
from jax import lax
from jax.experimental import pallas as pl
from jax.experimental.pallas import tpu as pltpu

_BN_EPS = 1e-5
_LANES = 128


def _pick_row_tile(h):
    for cand in (24, 16, 12, 8, 6, 4, 3, 2, 1):
        if h % cand == 0:
            return cand
    return 1


def _make_pass1_body(TR, W, Cin, Cpad):
    TR2 = 2 * TR
    R2 = TR2 + 2

    def body(x_ref, w_ref, y_ref, stats_ref, acc_ref):
        t = pl.program_id(1)
        t_last = pl.num_programs(1) - 1

        xb = x_ref[...].astype(jnp.float32)

        lo = xb[:-1]
        hi = xb[1:]
        ev = lo * 0.75 + hi * 0.25
        od = lo * 0.25 + hi * 0.75
        ru = jnp.stack([ev, od], axis=1).reshape(R2, W + 2, Cin)

        row = lax.broadcasted_iota(jnp.int32, (R2, 1, 1), 0)
        edge = jnp.logical_or(jnp.logical_and(t == 0, row == 0),
                              jnp.logical_and(t == t_last, row == R2 - 1))
        ru = jnp.where(edge, 0.0, ru)

        ca = ru[:, :W + 1]
        cb = ru[:, 1:]
        ue = ca * 0.25 + cb * 0.75
        uo = ca * 0.75 + cb * 0.25
        col = lax.broadcasted_iota(jnp.int32, (1, W + 1, 1), 1)
        ue = jnp.where(col == W, 0.0, ue).astype(jnp.bfloat16)
        uo = jnp.where(col == 0, 0.0, uo).astype(jnp.bfloat16)

        src_e, off_e = (uo, ue, uo), (0, 0, 1)
        src_o, off_o = (ue, uo, ue), (0, 1, 1)
        acc_e = jnp.zeros((TR2 * W, Cpad), jnp.float32)
        acc_o = jnp.zeros((TR2 * W, Cpad), jnp.float32)
        for dy in range(3):
            for dx in range(3):
                wt = w_ref[dy * 3 + dx]
                we = src_e[dx][dy:dy + TR2, off_e[dx]:off_e[dx] + W]
                wo = src_o[dx][dy:dy + TR2, off_o[dx]:off_o[dx] + W]
                acc_e += jnp.dot(we.reshape(TR2 * W, Cin), wt,
                                 preferred_element_type=jnp.float32)
                acc_o += jnp.dot(wo.reshape(TR2 * W, Cin), wt,
                                 preferred_element_type=jnp.float32)

        @pl.when(t == 0)
        def _():
            acc_ref[...] = jnp.zeros_like(acc_ref)

        acc_ref[0:1] += (jnp.sum(acc_e, axis=0, keepdims=True) +
                         jnp.sum(acc_o, axis=0, keepdims=True))
        acc_ref[1:2] += (jnp.sum(acc_e * acc_e, axis=0, keepdims=True) +
                         jnp.sum(acc_o * acc_o, axis=0, keepdims=True))

        @pl.when(t == t_last)
        def _():
            stats_ref[...] = acc_ref[...]

        y_ref[0] = acc_e.reshape(TR2, W, Cpad).astype(jnp.bfloat16)
        y_ref[1] = acc_o.reshape(TR2, W, Cpad).astype(jnp.bfloat16)

    return body


def _bn_act_body(y_ref, scale_ref, shift_ref, a_ref, o_ref):
    z = y_ref[...].astype(jnp.float32) * scale_ref[...] + shift_ref[...]
    slope = a_ref[0]
    o_ref[...] = jnp.where(z > 0, z, slope * z)


@jax.jit
def _forward(x_nchw, conv_w, conv_b, bn_gamma, bn_beta, prelu_a):
    del conv_b
    N, Cin, H, W = x_nchw.shape
    Cout = conv_w.shape[0]
    TR = _pick_row_tile(H)
    T = H // TR
    TR2 = 2 * TR
    Cpad = ((Cout + _LANES - 1) // _LANES) * _LANES

    x_nhwc = jnp.transpose(x_nchw.astype(jnp.bfloat16), (0, 2, 3, 1))
    xp = jnp.pad(x_nhwc, ((0, 0), (1, 1), (1, 1), (0, 0)), mode="edge")
    xt = jnp.stack([xp[:, t * TR:t * TR + TR + 2] for t in range(T)], axis=1)

    w9 = jnp.transpose(conv_w, (2, 3, 1, 0)).reshape(9, Cin, Cout)
    wp = jnp.pad(w9, ((0, 0), (0, 0), (0, Cpad - Cout))).astype(jnp.bfloat16)

    y, stats = pl.pallas_call(
        _make_pass1_body(TR, W, Cin, Cpad),
        out_shape=(
            jax.ShapeDtypeStruct((N, 2, 2 * H, W, Cpad), jnp.bfloat16),
            jax.ShapeDtypeStruct((N, 2, Cpad), jnp.float32),
        ),
        grid=(N, T),
        in_specs=[
            pl.BlockSpec((None, None, TR + 2, W + 2, Cin),
                         lambda n, t: (n, t, 0, 0, 0)),
            pl.BlockSpec((9, Cin, Cpad), lambda n, t: (0, 0, 0)),
        ],
        out_specs=(
            pl.BlockSpec((None, 2, TR2, W, Cpad), lambda n, t: (n, 0, t, 0, 0)),
            pl.BlockSpec((None, 2, Cpad), lambda n, t: (n, 0, 0)),
        ),
        scratch_shapes=[pltpu.VMEM((2, Cpad), jnp.float32)],
        compiler_params=pltpu.CompilerParams(
            dimension_semantics=("parallel", "arbitrary")),
    )(xt, wp)

    stot = jnp.sum(stats, axis=0)
    m_total = jnp.float32(N * (2 * H) * (2 * W))
    mean = stot[0] / m_total
    var = jnp.maximum(stot[1] / m_total - mean * mean, 0.0)
    gamma_p = jnp.pad(bn_gamma.astype(jnp.float32), (0, Cpad - Cout))
    beta_p = jnp.pad(bn_beta.astype(jnp.float32), (0, Cpad - Cout))
    scale = (gamma_p * lax.rsqrt(var + _BN_EPS)).reshape(1, Cpad)
    shift = (beta_p - mean * scale[0]).reshape(1, Cpad)
    a_smem = prelu_a.reshape(1).astype(jnp.float32)

    out = pl.pallas_call(
        _bn_act_body,
        out_shape=jax.ShapeDtypeStruct(y.shape, jnp.float32),
        grid=(N, T),
        in_specs=[
            pl.BlockSpec((None, 2, TR2, W, Cpad), lambda n, t: (n, 0, t, 0, 0)),
            pl.BlockSpec((1, Cpad), lambda n, t: (0, 0)),
            pl.BlockSpec((1, Cpad), lambda n, t: (0, 0)),
            pl.BlockSpec(memory_space=pltpu.MemorySpace.SMEM),
        ],
        out_specs=pl.BlockSpec((None, 2, TR2, W, Cpad),
                               lambda n, t: (n, 0, t, 0, 0)),
        compiler_params=pltpu.CompilerParams(
            dimension_semantics=("parallel", "parallel")),
    )(y, scale, shift, a_smem)

    out = out[..., :Cout]
    out = jnp.transpose(out, (0, 4, 2, 3, 1))
    return out.reshape(N, Cout, 2 * H, 2 * W)


def kernel(x_nchw, conv_w, conv_b, bn_gamma, bn_beta, prelu_a):
    return _forward(x_nchw, conv_w, conv_b, bn_gamma, bn_beta, prelu_a)
